# baseline (speedup 1.0000x reference)
import functools

import jax
import jax.numpy as jnp
from jax.experimental import pallas as pl
from jax.experimental.pallas import tpu as pltpu

R, C = 64, 100000
BC = 2048
GRID = (C + BC - 1) // BC


def _rotl(x, r):
    return (x << jnp.uint32(r)) | (x >> jnp.uint32(32 - r))


def _threefry_bits(n):
    k0 = jnp.uint32(0)
    k1 = jnp.uint32(42)
    ks = [k0, k1, jnp.uint32(0x1BD11BDA) ^ k0 ^ k1]
    rot_even = (13, 15, 26, 6)
    rot_odd = (17, 29, 16, 24)
    x0 = jnp.full_like(n, k0)
    x1 = n + k1
    for i in range(5):
        for r in rot_even if i % 2 == 0 else rot_odd:
            x0 = x0 + x1
            x1 = _rotl(x1, r)
            x1 = x1 ^ x0
        x0 = x0 + ks[(i + 1) % 3]
        x1 = x1 + ks[(i + 2) % 3] + jnp.uint32(i + 1)
    return x0 ^ x1


def _gumbel(rows, cols):
    n = (rows * C + cols).astype(jnp.uint32)
    bits = _threefry_bits(n)
    fb = (bits >> jnp.uint32(9)) | jnp.uint32(0x3F800000)
    u = jax.lax.bitcast_convert_type(fb, jnp.float32) - jnp.float32(1.0)
    eps = jnp.float32(1e-10)
    return -jnp.log(-jnp.log(u + eps) + eps)


def _argmax_body(x_ref, maxv_ref, argc_ref):
    step = pl.program_id(0)
    shape = (R, BC)
    cols = jax.lax.broadcasted_iota(jnp.int32, shape, 1) + step * BC
    rows = jax.lax.broadcasted_iota(jnp.int32, shape, 0)
    y = x_ref[...] + _gumbel(rows, cols)
    y = jnp.where(cols < C, y, -jnp.inf)
    m = jnp.max(y, axis=1, keepdims=True)
    cand = jnp.where(y == m, cols, jnp.int32(2**31 - 1))
    a = jnp.min(cand, axis=1, keepdims=True)

    @pl.when(step == 0)
    def _():
        maxv_ref[...] = m
        argc_ref[...] = a

    @pl.when(step > 0)
    def _():
        upd = m > maxv_ref[...]
        maxv_ref[...] = jnp.where(upd, m, maxv_ref[...])
        argc_ref[...] = jnp.where(upd, a, argc_ref[...])


def _onehot_body(argc_ref, o_ref):
    step = pl.program_id(0)
    cols = jax.lax.broadcasted_iota(jnp.int32, (R, BC), 1) + step * BC
    o_ref[...] = jnp.where(cols == argc_ref[...], jnp.float32(1.0),
                           jnp.float32(0.0))


@jax.jit
def kernel(logits):
    _, argc = pl.pallas_call(
        _argmax_body,
        grid=(GRID,),
        in_specs=[pl.BlockSpec((R, BC), lambda i: (0, i))],
        out_specs=[
            pl.BlockSpec((R, 1), lambda i: (0, 0)),
            pl.BlockSpec((R, 1), lambda i: (0, 0)),
        ],
        out_shape=[
            jax.ShapeDtypeStruct((R, 1), jnp.float32),
            jax.ShapeDtypeStruct((R, 1), jnp.int32),
        ],
        compiler_params=pltpu.CompilerParams(
            dimension_semantics=("arbitrary",)),
    )(logits)
    out = pl.pallas_call(
        _onehot_body,
        grid=(GRID,),
        in_specs=[pl.BlockSpec((R, 1), lambda i: (0, 0))],
        out_specs=pl.BlockSpec((R, BC), lambda i: (0, i)),
        out_shape=jax.ShapeDtypeStruct((R, C), jnp.float32),
        compiler_params=pltpu.CompilerParams(
            dimension_semantics=("arbitrary",)),
    )(argc)
    return out
